# Initial kernel scaffold; baseline (speedup 1.0000x reference)
import jax
import jax.numpy as jnp
from jax.experimental import pallas as pl


def kernel(atom_feats, bond_attr, triplet_feats, edge_index, angle_index, ba_W, ba_att, ba_bias, ba_bnn_g, ba_bnn_b, ba_lf_w, ba_lf_b, ba_ls_w, ba_ls_b, ba_bne_g, ba_bne_b, bb_W, bb_att, bb_bias, bb_bnn_g, bb_bnn_b, bb_lf_w, bb_lf_b, bb_ls_w, bb_ls_b, bb_bne_g, bb_bne_b):
    raise NotImplementedError("write your pallas kernel here")


# speedup vs baseline: 16.0054x; 16.0054x over previous
import functools

import jax
import jax.numpy as jnp
from jax import lax
from jax.experimental import pallas as pl
from jax.experimental.pallas import tpu as pltpu
from jax.experimental.pallas import tpu_sc as plsc

NC = 2
NS = 16
LANES = 16
NW = NC * NS

DIM = 128
HEADS = 4
HP = 8

_F32 = jnp.float32


def _mesh():
  return plsc.VectorSubcoreMesh(
      core_axis_name="c", subcore_axis_name="s", num_cores=NC,
      num_subcores=NS)


_SC_PARAMS = pltpu.CompilerParams(use_tc_tiling_on_sc=False)


def _sc_gather2(table, i0, i1):
  eg = i0.shape[0]
  v, d = table.shape
  per = eg // NW
  c = 200
  n_chunks = per // c

  @functools.partial(
      pl.kernel,
      out_type=[jax.ShapeDtypeStruct((eg, d), _F32),
                jax.ShapeDtypeStruct((eg, d), _F32)],
      mesh=_mesh(),
      compiler_params=_SC_PARAMS,
      scratch_types=[pltpu.VMEM((per,), jnp.int32),
                     pltpu.VMEM((c, d), _F32),
                     pltpu.VMEM((c, d), _F32),
                     pltpu.SemaphoreType.DMA,
                     pltpu.SemaphoreType.DMA],
  )
  def k(table_h, i0_h, i1_h, o0_h, o1_h, idx_v, rows0_v, rows1_v, sem0, sem1):
    wid = lax.axis_index("s") * NC + lax.axis_index("c")
    base = wid * per
    for which in range(2):
      src_idx = (i0_h, i1_h)[which]
      dst = (o0_h, o1_h)[which]
      rows_v = (rows0_v, rows1_v)[which]
      sem = (sem0, sem1)[which]
      pltpu.sync_copy(src_idx.at[pl.ds(base, per)], idx_v)

      def body(ci, _, dst=dst, rows_v=rows_v, sem=sem):
        off = ci * c
        pltpu.async_copy(
            table_h.at[idx_v.at[pl.ds(off, c)]], rows_v, sem).wait()
        pltpu.sync_copy(rows_v, dst.at[pl.ds(base + off, c)])
        return 0

      lax.fori_loop(0, n_chunks, body, 0, unroll=False)

  return k(table, i0, i1)


def _sc_seg_softmax(alpha8, idx, ctab, n):
  eg = alpha8.shape[0] // HP
  per = eg // NS
  ec = 2000
  n_chunks = per // ec
  vper = ec // LANES
  nz = (-(-n // NS) + 7) // 8 * 8
  n_pad = NS * nz
  zc = 2000 if nz % 2000 == 0 else nz
  assert zc <= ec and nz % zc == 0

  @functools.partial(
      pl.kernel,
      out_type=jax.ShapeDtypeStruct((HEADS * eg,), _F32),
      mesh=_mesh(),
      compiler_params=_SC_PARAMS,
      scratch_types=[pltpu.VMEM((ec,), _F32),
                     pltpu.VMEM((2 * per,), _F32),
                     pltpu.VMEM((ec,), _F32),
                     pltpu.VMEM((ec,), _F32),
                     pltpu.VMEM((ec,), _F32),
                     pltpu.VMEM((256,), _F32),
                     pltpu.VMEM_SHARED((2 * n_pad,), _F32),
                     pltpu.SemaphoreType.DMA]
      + [pltpu.VMEM((ec,), jnp.int32) for _ in range(n_chunks)],
  )
  def k(alpha_h, idx_h, ctab_h, w_h,
        ach_v, ex_v, den_v, wch_v, z_v, ct_v, den_s, sem, *idxs):
    core = lax.axis_index("c")
    tile = lax.axis_index("s")
    e0 = tile * per
    pltpu.sync_copy(ctab_h, ct_v)
    for ck in range(n_chunks):
      pltpu.sync_copy(idx_h.at[pl.ds(e0 + ck * ec, ec)], idxs[ck])

    def zfill(i, _):
      z_v[pl.ds(i * LANES, LANES)] = jnp.zeros((LANES,), _F32)
      return 0
    lax.fori_loop(0, ec // LANES, zfill, 0, unroll=False)
    r0 = tile * nz

    def zden(i, _):
      pltpu.sync_copy(z_v.at[pl.ds(0, zc)],
                      den_s.at[pl.ds(r0 + i * zc, zc)])
      pltpu.sync_copy(z_v.at[pl.ds(0, zc)],
                      den_s.at[pl.ds(n_pad + r0 + i * zc, zc)])
      return 0
    lax.fori_loop(0, nz // zc, zden, 0, unroll=False)
    plsc.subcore_barrier()

    for c_val in range(NC):
      @pl.when(core == c_val)
      def _():
        for hl in range(2):
          h = 2 * c_val + hl
          scv = ct_v[pl.ds(h * 16, 16)]
          shv = ct_v[pl.ds(64 + h * 16, 16)]
          gmv = ct_v[pl.ds(128 + h * 16, 16)]
          for ck in range(n_chunks):
            pltpu.sync_copy(
                alpha_h.at[pl.ds(h * eg + e0 + ck * ec, ec)], ach_v)
            exo = (hl * n_chunks + ck) * ec

            def exbody(l, _, scv=scv, shv=shv, gmv=gmv, exo=exo):
              a = ach_v[pl.ds(l * LANES, LANES)]
              an = a * scv + shv
              sig = 1.0 / (1.0 + jnp.exp(-an))
              ex = jnp.exp(an * sig - gmv)
              ex_v[pl.ds(exo + l * LANES, LANES)] = ex
              return 0
            lax.fori_loop(0, vper, exbody, 0, unroll=False)
            pltpu.sync_copy(
                ex_v.at[pl.ds(exo, ec)],
                den_s.at[pl.ds(hl * n_pad, n_pad)].at[idxs[ck]], add=True)
        plsc.subcore_barrier()
        for hl in range(2):
          h = 2 * c_val + hl
          for ck in range(n_chunks):
            pltpu.async_copy(
                den_s.at[pl.ds(hl * n_pad, n_pad)].at[idxs[ck]],
                den_v, sem).wait()
            exo = (hl * n_chunks + ck) * ec

            def wbody(l, _, exo=exo):
              ex = ex_v[pl.ds(exo + l * LANES, LANES)]
              dv = den_v[pl.ds(l * LANES, LANES)]
              wch_v[pl.ds(l * LANES, LANES)] = (
                  ex / (dv + 1e-16) * (1.0 / HEADS))
              return 0
            lax.fori_loop(0, vper, wbody, 0, unroll=False)
            pltpu.sync_copy(wch_v, w_h.at[pl.ds(h * eg + e0 + ck * ec, ec)])

  return k(alpha8, idx, ctab)


def _sc_scatter_aggr(msum, idx, n):
  eg = idx.shape[0]
  per = eg // NS
  g = 8 if n > 20000 else 16
  npass = (DIM // NC) // g
  ec = 2000 if g == 8 else 1000
  n_chunks = per // ec
  nr = (-(-n // NS) + 7) // 8 * 8
  n_pad = NS * nr
  zr = 1000 if nr % 1000 == 0 else nr
  wbr = 2000 if nr % 2000 == 0 else nr
  assert nr % zr == 0 and nr % wbr == 0

  zeros_in = jnp.zeros((zr, g), _F32)

  @functools.partial(
      pl.kernel,
      out_type=jax.ShapeDtypeStruct((n_pad, DIM), _F32),
      mesh=_mesh(),
      compiler_params=_SC_PARAMS,
      scratch_types=[pltpu.VMEM((ec, g), _F32),
                     pltpu.VMEM((zr, g), _F32),
                     pltpu.VMEM((wbr, g), _F32),
                     pltpu.VMEM_SHARED((n_pad, g), _F32)]
      + [pltpu.VMEM((ec,), jnp.int32) for _ in range(n_chunks)],
  )
  def k(msum_h, idx_h, z_h, aggr_h, upd_v, z_v, wb_v, acc_s, *idxs):
    core = lax.axis_index("c")
    tile = lax.axis_index("s")
    e0 = tile * per
    for ck in range(n_chunks):
      pltpu.sync_copy(idx_h.at[pl.ds(e0 + ck * ec, ec)], idxs[ck])
    pltpu.sync_copy(z_h, z_v)
    r0 = tile * nr

    for c_val in range(NC):
      @pl.when(core == c_val)
      def _():
        for p in range(npass):
          c0 = (DIM // NC) * c_val + p * g

          def zacc(i, _):
            pltpu.sync_copy(z_v, acc_s.at[pl.ds(r0 + i * zr, zr)])
            return 0
          lax.fori_loop(0, nr // zr, zacc, 0, unroll=False)
          plsc.subcore_barrier()
          for ck in range(n_chunks):
            pltpu.sync_copy(
                msum_h.at[pl.ds(e0 + ck * ec, ec), pl.ds(c0, g)], upd_v)
            pltpu.sync_copy(upd_v, acc_s.at[idxs[ck]], add=True)
          plsc.subcore_barrier()

          def wback(i, _, c0=c0):
            pltpu.sync_copy(acc_s.at[pl.ds(r0 + i * wbr, wbr)], wb_v)
            pltpu.sync_copy(
                wb_v, aggr_h.at[pl.ds(r0 + i * wbr, wbr), pl.ds(c0, g)])
            return 0
          lax.fori_loop(0, nr // wbr, wback, 0, unroll=False)

  return k(msum, idx, zeros_in)


def _tc_edge_attn(xi, xj, ea, w1, w2, att1_8, att2_8, g8, b8):
  eg = xi.shape[0]
  be = 1280
  nb = eg // be

  def body(xi_r, xj_r, ea_r, w1_r, w2_r, a1_r, a2_r, g_r, b_r,
           njq_r, alpha_r, stats_r, acc_r):
    step = pl.program_id(0)
    tsh = jnp.dot(ea_r[...], w2_r[...], preferred_element_type=_F32)
    ti = jnp.dot(xi_r[...], w1_r[...], preferred_element_type=_F32) + tsh
    tj = jnp.dot(xj_r[...], w1_r[...], preferred_element_type=_F32) + tsh
    ni = ti * jax.nn.sigmoid(ti)
    nj = tj * jax.nn.sigmoid(tj)
    njq_r[...] = nj.astype(jnp.bfloat16)
    pre = (jnp.dot(ni, a1_r[...], preferred_element_type=_F32)
           + jnp.dot(nj, a2_r[...], preferred_element_type=_F32))
    alpha = pre * jax.nn.sigmoid(pre)
    alpha_r[...] = alpha.T
    s = jnp.sum(alpha, axis=0, keepdims=True)
    sq = jnp.sum(alpha * alpha, axis=0, keepdims=True)
    mn = jnp.min(alpha, axis=0, keepdims=True)
    mx = jnp.max(alpha, axis=0, keepdims=True)

    @pl.when(step == 0)
    def _():
      acc_r[0:1, :] = s
      acc_r[1:2, :] = sq
      acc_r[2:3, :] = mn
      acc_r[3:4, :] = mx

    @pl.when(step > 0)
    def _():
      acc_r[0:1, :] = acc_r[0:1, :] + s
      acc_r[1:2, :] = acc_r[1:2, :] + sq
      acc_r[2:3, :] = jnp.minimum(acc_r[2:3, :], mn)
      acc_r[3:4, :] = jnp.maximum(acc_r[3:4, :], mx)

    @pl.when(step == nb - 1)
    def _():
      mu = acc_r[0:1, :] / eg
      var = acc_r[1:2, :] / eg - mu * mu
      sc = g_r[...] * lax.rsqrt(var + 1e-5)
      sh = b_r[...] - mu * sc
      lo = acc_r[2:3, :] * sc + sh
      hi = acc_r[3:4, :] * sc + sh
      lo2 = jnp.minimum(lo, hi)
      hi2 = jnp.maximum(lo, hi)
      gm = jnp.maximum(lo2 * jax.nn.sigmoid(lo2), hi2 * jax.nn.sigmoid(hi2))
      stats_r[0:1, :] = sc
      stats_r[1:2, :] = sh
      stats_r[2:3, :] = gm
      stats_r[3:4, :] = jnp.zeros_like(gm)

  out = pl.pallas_call(
      body,
      grid=(nb,),
      in_specs=[
          pl.BlockSpec((be, DIM), lambda i: (i, 0)),
          pl.BlockSpec((be, DIM), lambda i: (i, 0)),
          pl.BlockSpec((be, DIM), lambda i: (i, 0)),
          pl.BlockSpec((DIM, HEADS * DIM), lambda i: (0, 0)),
          pl.BlockSpec((DIM, HEADS * DIM), lambda i: (0, 0)),
          pl.BlockSpec((HEADS * DIM, HP), lambda i: (0, 0)),
          pl.BlockSpec((HEADS * DIM, HP), lambda i: (0, 0)),
          pl.BlockSpec((1, HP), lambda i: (0, 0)),
          pl.BlockSpec((1, HP), lambda i: (0, 0)),
      ],
      out_specs=[
          pl.BlockSpec((be, HEADS * DIM), lambda i: (i, 0)),
          pl.BlockSpec((HP, be), lambda i: (0, i)),
          pl.BlockSpec((4, HP), lambda i: (0, 0)),
      ],
      out_shape=[
          jax.ShapeDtypeStruct((eg, HEADS * DIM), jnp.bfloat16),
          jax.ShapeDtypeStruct((HP, eg), _F32),
          jax.ShapeDtypeStruct((4, HP), _F32),
      ],
      scratch_shapes=[pltpu.VMEM((4, HP), _F32)],
  )(xi, xj, ea, w1, w2, att1_8, att2_8, g8, b8)
  return out


def _tc_msum(njq, wt4):
  eg = njq.shape[0]
  be = 1280
  nb = eg // be

  def body(nj_r, wt_r, out_r):
    nj = nj_r[...].astype(_F32)
    w4 = wt_r[...].T
    acc = nj[:, 0:DIM] * w4[:, 0:1]
    for h in range(1, HEADS):
      acc = acc + nj[:, h * DIM:(h + 1) * DIM] * w4[:, h:h + 1]
    out_r[...] = acc

  return pl.pallas_call(
      body,
      grid=(nb,),
      in_specs=[
          pl.BlockSpec((be, HEADS * DIM), lambda i: (i, 0)),
          pl.BlockSpec((HEADS, be), lambda i: (0, i)),
      ],
      out_specs=pl.BlockSpec((be, DIM), lambda i: (i, 0)),
      out_shape=jax.ShapeDtypeStruct((eg, DIM), _F32),
  )(njq, wt4)


def _tc_edge_update(nfi, nfj, ea, lf1, lf2, lf3, lfb, ls1, ls2, ls3, lsb,
                    g_row, b_row):
  eg = nfi.shape[0]
  be = 2000
  nb = eg // be

  def body(nfi_r, nfj_r, ea_r, lf1_r, lf2_r, lf3_r, lfb_r,
           ls1_r, ls2_r, ls3_r, lsb_r, g_r, b_r, m_r, stats_r, acc_r):
    step = pl.program_id(0)
    zi = (jnp.dot(nfi_r[...], lf1_r[...], preferred_element_type=_F32)
          + jnp.dot(nfj_r[...], lf2_r[...], preferred_element_type=_F32)
          + jnp.dot(ea_r[...], lf3_r[...], preferred_element_type=_F32)
          + lfb_r[...])
    zs = (jnp.dot(nfi_r[...], ls1_r[...], preferred_element_type=_F32)
          + jnp.dot(nfj_r[...], ls2_r[...], preferred_element_type=_F32)
          + jnp.dot(ea_r[...], ls3_r[...], preferred_element_type=_F32)
          + lsb_r[...])
    m = jax.nn.sigmoid(zi) * jax.nn.softplus(zs)
    m_r[...] = m
    s = jnp.sum(m, axis=0, keepdims=True)
    sq = jnp.sum(m * m, axis=0, keepdims=True)

    @pl.when(step == 0)
    def _():
      acc_r[0:1, :] = s
      acc_r[1:2, :] = sq

    @pl.when(step > 0)
    def _():
      acc_r[0:1, :] = acc_r[0:1, :] + s
      acc_r[1:2, :] = acc_r[1:2, :] + sq

    @pl.when(step == nb - 1)
    def _():
      mu = acc_r[0:1, :] / eg
      var = acc_r[1:2, :] / eg - mu * mu
      sc = g_r[...] * lax.rsqrt(var + 1e-5)
      stats_r[0:1, :] = sc
      stats_r[1:2, :] = b_r[...] - mu * sc

  return pl.pallas_call(
      body,
      grid=(nb,),
      in_specs=[pl.BlockSpec((be, DIM), lambda i: (i, 0))] * 3
      + [pl.BlockSpec((DIM, DIM), lambda i: (0, 0)),
         pl.BlockSpec((DIM, DIM), lambda i: (0, 0)),
         pl.BlockSpec((DIM, DIM), lambda i: (0, 0)),
         pl.BlockSpec((1, DIM), lambda i: (0, 0))] * 2
      + [pl.BlockSpec((1, DIM), lambda i: (0, 0)),
         pl.BlockSpec((1, DIM), lambda i: (0, 0))],
      out_specs=[
          pl.BlockSpec((be, DIM), lambda i: (i, 0)),
          pl.BlockSpec((2, DIM), lambda i: (0, 0)),
      ],
      out_shape=[
          jax.ShapeDtypeStruct((eg, DIM), _F32),
          jax.ShapeDtypeStruct((2, DIM), _F32),
      ],
      scratch_shapes=[pltpu.VMEM((2, DIM), _F32)],
  )(nfi, nfj, ea, lf1, lf2, lf3, lfb, ls1, ls2, ls3, lsb, g_row, b_row)


def _tc_axpb(base, m, scale_row, shift_row):
  n = base.shape[0]
  be = 2000
  nb = n // be

  def body(b_r, m_r, sc_r, sh_r, o_r):
    o_r[...] = b_r[...] + m_r[...] * sc_r[...] + sh_r[...]

  return pl.pallas_call(
      body,
      grid=(nb,),
      in_specs=[
          pl.BlockSpec((be, DIM), lambda i: (i, 0)),
          pl.BlockSpec((be, DIM), lambda i: (i, 0)),
          pl.BlockSpec((1, DIM), lambda i: (0, 0)),
          pl.BlockSpec((1, DIM), lambda i: (0, 0)),
      ],
      out_specs=pl.BlockSpec((be, DIM), lambda i: (i, 0)),
      out_shape=jax.ShapeDtypeStruct((n, DIM), _F32),
  )(base, m, scale_row, shift_row)


def _gcao_layer(x, edge_index, ea, w, att, bias, bnn_g, bnn_b,
                lf_w, lf_b, ls_w, ls_b, bne_g, bne_b):
  n = x.shape[0]
  idx_i = edge_index[0]
  idx_j = edge_index[1]

  w1 = w[:DIM]
  w2 = w[DIM:]
  a1 = att[0, :, :DIM]
  a2 = att[0, :, DIM:]
  eye = jnp.eye(HEADS, HP, dtype=_F32)
  att1_8 = jnp.einsum("hd,hp->hdp", a1, eye).reshape(HEADS * DIM, HP)
  att2_8 = jnp.einsum("hd,hp->hdp", a2, eye).reshape(HEADS * DIM, HP)
  g8 = jnp.concatenate([bnn_g, jnp.ones((HP - HEADS,), _F32)])[None, :]
  b8 = jnp.concatenate([bnn_b, jnp.zeros((HP - HEADS,), _F32)])[None, :]

  xi, xj = _sc_gather2(x, idx_i, idx_j)
  njq, alpha8, stats = _tc_edge_attn(xi, xj, ea, w1, w2, att1_8, att2_8,
                                     g8, b8)
  ctab = jnp.zeros((16, 16), _F32).at[0:12, :].set(
      jnp.repeat(stats[0:3, 0:HEADS].reshape(12, 1), 16, axis=1)).reshape(256)
  eg = njq.shape[0]
  w_f = _sc_seg_softmax(alpha8.reshape(-1), idx_i, ctab, n)
  msum = _tc_msum(njq, w_f.reshape(HEADS, eg))
  aggr = _sc_scatter_aggr(msum, idx_i, n)
  ones_row = jnp.ones((1, DIM), _F32)
  node_feat = _tc_axpb(x, aggr, ones_row, bias[None, :])

  nfi, nfj = _sc_gather2(node_feat, idx_i, idx_j)
  lf1 = lf_w[:, 0:DIM].T
  lf2 = lf_w[:, DIM:2 * DIM].T
  lf3 = lf_w[:, 2 * DIM:].T
  ls1 = ls_w[:, 0:DIM].T
  ls2 = ls_w[:, DIM:2 * DIM].T
  ls3 = ls_w[:, 2 * DIM:].T
  m_raw, mstats = _tc_edge_update(nfi, nfj, ea, lf1, lf2, lf3, lf_b[None, :],
                                  ls1, ls2, ls3, ls_b[None, :],
                                  bne_g[None, :], bne_b[None, :])
  edge_feat = _tc_axpb(ea, m_raw, mstats[0:1, :], mstats[1:2, :])
  return node_feat, edge_feat


def kernel(atom_feats, bond_attr, triplet_feats, edge_index, angle_index,
           ba_W, ba_att, ba_bias, ba_bnn_g, ba_bnn_b, ba_lf_w, ba_lf_b,
           ba_ls_w, ba_ls_b, ba_bne_g, ba_bne_b,
           bb_W, bb_att, bb_bias, bb_bnn_g, bb_bnn_b, bb_lf_w, bb_lf_b,
           bb_ls_w, bb_ls_b, bb_bne_g, bb_bne_b):
  bond, triplet = _gcao_layer(bond_attr, angle_index, triplet_feats,
                              ba_W, ba_att, ba_bias, ba_bnn_g, ba_bnn_b,
                              ba_lf_w, ba_lf_b, ba_ls_w, ba_ls_b,
                              ba_bne_g, ba_bne_b)
  atom, bond2 = _gcao_layer(atom_feats, edge_index, bond,
                            bb_W, bb_att, bb_bias, bb_bnn_g, bb_bnn_b,
                            bb_lf_w, bb_lf_b, bb_ls_w, bb_ls_b,
                            bb_bne_g, bb_bne_b)
  return (atom, bond2, triplet)


# speedup vs baseline: 16.1074x; 1.0064x over previous
import functools

import jax
import jax.numpy as jnp
from jax import lax
from jax.experimental import pallas as pl
from jax.experimental.pallas import tpu as pltpu
from jax.experimental.pallas import tpu_sc as plsc

NC = 2
NS = 16
LANES = 16
NW = NC * NS

DIM = 128
HEADS = 4
HP = 8

_F32 = jnp.float32


def _mesh():
  return plsc.VectorSubcoreMesh(
      core_axis_name="c", subcore_axis_name="s", num_cores=NC,
      num_subcores=NS)


_SC_PARAMS = pltpu.CompilerParams(use_tc_tiling_on_sc=False)


def _sc_gather2(table, i0, i1):
  eg = i0.shape[0]
  v, d = table.shape
  per = eg // NW
  c = 200
  n_chunks = per // c

  @functools.partial(
      pl.kernel,
      out_type=[jax.ShapeDtypeStruct((eg, d), _F32),
                jax.ShapeDtypeStruct((eg, d), _F32)],
      mesh=_mesh(),
      compiler_params=_SC_PARAMS,
      scratch_types=[pltpu.VMEM((per,), jnp.int32),
                     pltpu.VMEM((per,), jnp.int32),
                     pltpu.VMEM((c, d), _F32),
                     pltpu.VMEM((c, d), _F32),
                     pltpu.SemaphoreType.DMA,
                     pltpu.SemaphoreType.DMA],
  )
  def k(table_h, i0_h, i1_h, o0_h, o1_h,
        idx0_v, idx1_v, rows0_v, rows1_v, sem0, sem1):
    wid = lax.axis_index("s") * NC + lax.axis_index("c")
    base = wid * per
    pltpu.sync_copy(i0_h.at[pl.ds(base, per)], idx0_v)
    pltpu.sync_copy(i1_h.at[pl.ds(base, per)], idx1_v)
    bufs = (rows0_v, rows1_v)
    sems = (sem0, sem1)
    chunks = ([(idx0_v, o0_h, ci) for ci in range(n_chunks)]
              + [(idx1_v, o1_h, ci) for ci in range(n_chunks)])
    pend = [None, None]
    for kk, (iv, dst, ci) in enumerate(chunks):
      b = kk % 2
      if pend[b] is not None:
        cp, pdst, poff = pend[b]
        cp.wait()
        pltpu.sync_copy(bufs[b], pdst.at[pl.ds(base + poff, c)])
      cp = pltpu.async_copy(
          table_h.at[iv.at[pl.ds(ci * c, c)]], bufs[b], sems[b])
      pend[b] = (cp, dst, ci * c)
    for b in range(2):
      cp, pdst, poff = pend[b]
      cp.wait()
      pltpu.sync_copy(bufs[b], pdst.at[pl.ds(base + poff, c)])

  return k(table, i0, i1)


def _sc_seg_softmax(alpha8, idx, ctab, n):
  eg = alpha8.shape[0] // HP
  per = eg // NS
  ec = 2000
  n_chunks = per // ec
  vper = ec // LANES
  nz = (-(-n // NS) + 7) // 8 * 8
  n_pad = NS * nz
  zc = 2000 if nz % 2000 == 0 else nz
  assert zc <= ec and nz % zc == 0

  @functools.partial(
      pl.kernel,
      out_type=jax.ShapeDtypeStruct((HEADS * eg,), _F32),
      mesh=_mesh(),
      compiler_params=_SC_PARAMS,
      scratch_types=[pltpu.VMEM((ec,), _F32),
                     pltpu.VMEM((2 * per,), _F32),
                     pltpu.VMEM((ec,), _F32),
                     pltpu.VMEM((ec,), _F32),
                     pltpu.VMEM((ec,), _F32),
                     pltpu.VMEM((256,), _F32),
                     pltpu.VMEM_SHARED((2 * n_pad,), _F32),
                     pltpu.SemaphoreType.DMA]
      + [pltpu.VMEM((ec,), jnp.int32) for _ in range(n_chunks)],
  )
  def k(alpha_h, idx_h, ctab_h, w_h,
        ach_v, ex_v, den_v, wch_v, z_v, ct_v, den_s, sem, *idxs):
    core = lax.axis_index("c")
    tile = lax.axis_index("s")
    e0 = tile * per
    pltpu.sync_copy(ctab_h, ct_v)
    for ck in range(n_chunks):
      pltpu.sync_copy(idx_h.at[pl.ds(e0 + ck * ec, ec)], idxs[ck])

    def zfill(i, _):
      z_v[pl.ds(i * LANES, LANES)] = jnp.zeros((LANES,), _F32)
      return 0
    lax.fori_loop(0, ec // LANES, zfill, 0, unroll=False)
    r0 = tile * nz

    def zden(i, _):
      pltpu.sync_copy(z_v.at[pl.ds(0, zc)],
                      den_s.at[pl.ds(r0 + i * zc, zc)])
      pltpu.sync_copy(z_v.at[pl.ds(0, zc)],
                      den_s.at[pl.ds(n_pad + r0 + i * zc, zc)])
      return 0
    lax.fori_loop(0, nz // zc, zden, 0, unroll=False)
    plsc.subcore_barrier()

    for c_val in range(NC):
      @pl.when(core == c_val)
      def _():
        for hl in range(2):
          h = 2 * c_val + hl
          scv = ct_v[pl.ds(h * 16, 16)]
          shv = ct_v[pl.ds(64 + h * 16, 16)]
          gmv = ct_v[pl.ds(128 + h * 16, 16)]
          for ck in range(n_chunks):
            pltpu.sync_copy(
                alpha_h.at[pl.ds(h * eg + e0 + ck * ec, ec)], ach_v)
            exo = (hl * n_chunks + ck) * ec

            def exbody(l, _, scv=scv, shv=shv, gmv=gmv, exo=exo):
              a = ach_v[pl.ds(l * LANES, LANES)]
              an = a * scv + shv
              sig = 1.0 / (1.0 + jnp.exp(-an))
              ex = jnp.exp(an * sig - gmv)
              ex_v[pl.ds(exo + l * LANES, LANES)] = ex
              return 0
            lax.fori_loop(0, vper, exbody, 0, unroll=False)
            pltpu.sync_copy(
                ex_v.at[pl.ds(exo, ec)],
                den_s.at[pl.ds(hl * n_pad, n_pad)].at[idxs[ck]], add=True)
        plsc.subcore_barrier()
        for hl in range(2):
          h = 2 * c_val + hl
          for ck in range(n_chunks):
            pltpu.async_copy(
                den_s.at[pl.ds(hl * n_pad, n_pad)].at[idxs[ck]],
                den_v, sem).wait()
            exo = (hl * n_chunks + ck) * ec

            def wbody(l, _, exo=exo):
              ex = ex_v[pl.ds(exo + l * LANES, LANES)]
              dv = den_v[pl.ds(l * LANES, LANES)]
              wch_v[pl.ds(l * LANES, LANES)] = (
                  ex / (dv + 1e-16) * (1.0 / HEADS))
              return 0
            lax.fori_loop(0, vper, wbody, 0, unroll=False)
            pltpu.sync_copy(wch_v, w_h.at[pl.ds(h * eg + e0 + ck * ec, ec)])

  return k(alpha8, idx, ctab)


def _sc_scatter_aggr(msum, idx, n):
  eg = idx.shape[0]
  per = eg // NS
  g = 8
  npass = (DIM // NC) // g
  ec = 2000
  n_chunks = per // ec
  nr = (-(-n // NS) + 7) // 8 * 8
  n_pad = NS * nr
  zr = 1000 if nr % 1000 == 0 else nr
  wbr = 2000 if nr % 2000 == 0 else nr
  assert nr % zr == 0 and nr % wbr == 0

  zeros_in = jnp.zeros((zr, g), _F32)

  @functools.partial(
      pl.kernel,
      out_type=jax.ShapeDtypeStruct((n_pad, DIM), _F32),
      mesh=_mesh(),
      compiler_params=_SC_PARAMS,
      scratch_types=[pltpu.VMEM((ec, g), _F32),
                     pltpu.VMEM((ec, g), _F32),
                     pltpu.VMEM((zr, g), _F32),
                     pltpu.VMEM((wbr, g), _F32),
                     pltpu.VMEM_SHARED((n_pad, g), _F32),
                     pltpu.SemaphoreType.DMA,
                     pltpu.SemaphoreType.DMA]
      + [pltpu.VMEM((ec,), jnp.int32) for _ in range(n_chunks)],
  )
  def k(msum_h, idx_h, z_h, aggr_h, upd0_v, upd1_v, z_v, wb_v, acc_s,
        ssem0, ssem1, *idxs):
    core = lax.axis_index("c")
    tile = lax.axis_index("s")
    e0 = tile * per
    for ck in range(n_chunks):
      pltpu.sync_copy(idx_h.at[pl.ds(e0 + ck * ec, ec)], idxs[ck])
    pltpu.sync_copy(z_h, z_v)
    r0 = tile * nr

    for c_val in range(NC):
      @pl.when(core == c_val)
      def _():
        for p in range(npass):
          c0 = (DIM // NC) * c_val + p * g

          def zacc(i, _):
            pltpu.sync_copy(z_v, acc_s.at[pl.ds(r0 + i * zr, zr)])
            return 0
          lax.fori_loop(0, nr // zr, zacc, 0, unroll=False)
          plsc.subcore_barrier()
          for ck in range(n_chunks):
            pltpu.sync_copy(
                msum_h.at[pl.ds(e0 + ck * ec, ec), pl.ds(c0, g)], upd0_v)
            pltpu.sync_copy(upd0_v, acc_s.at[idxs[ck]], add=True)
          plsc.subcore_barrier()

          def wback(i, _, c0=c0):
            pltpu.sync_copy(acc_s.at[pl.ds(r0 + i * wbr, wbr)], wb_v)
            pltpu.sync_copy(
                wb_v, aggr_h.at[pl.ds(r0 + i * wbr, wbr), pl.ds(c0, g)])
            return 0
          lax.fori_loop(0, nr // wbr, wback, 0, unroll=False)

  return k(msum, idx, zeros_in)


def _tc_edge_attn(xi, xj, ea, w1, w2, att1_8, att2_8, g8, b8):
  eg = xi.shape[0]
  be = 1280
  nb = eg // be

  def body(xi_r, xj_r, ea_r, w1_r, w2_r, a1_r, a2_r, g_r, b_r,
           njq_r, alpha_r, stats_r, acc_r):
    step = pl.program_id(0)
    tsh = jnp.dot(ea_r[...], w2_r[...], preferred_element_type=_F32)
    ti = jnp.dot(xi_r[...], w1_r[...], preferred_element_type=_F32) + tsh
    tj = jnp.dot(xj_r[...], w1_r[...], preferred_element_type=_F32) + tsh
    ni = ti * jax.nn.sigmoid(ti)
    nj = tj * jax.nn.sigmoid(tj)
    njq_r[...] = nj.astype(jnp.bfloat16)
    pre = (jnp.dot(ni, a1_r[...], preferred_element_type=_F32)
           + jnp.dot(nj, a2_r[...], preferred_element_type=_F32))
    alpha = pre * jax.nn.sigmoid(pre)
    alpha_r[...] = alpha.T
    s = jnp.sum(alpha, axis=0, keepdims=True)
    sq = jnp.sum(alpha * alpha, axis=0, keepdims=True)
    mn = jnp.min(alpha, axis=0, keepdims=True)
    mx = jnp.max(alpha, axis=0, keepdims=True)

    @pl.when(step == 0)
    def _():
      acc_r[0:1, :] = s
      acc_r[1:2, :] = sq
      acc_r[2:3, :] = mn
      acc_r[3:4, :] = mx

    @pl.when(step > 0)
    def _():
      acc_r[0:1, :] = acc_r[0:1, :] + s
      acc_r[1:2, :] = acc_r[1:2, :] + sq
      acc_r[2:3, :] = jnp.minimum(acc_r[2:3, :], mn)
      acc_r[3:4, :] = jnp.maximum(acc_r[3:4, :], mx)

    @pl.when(step == nb - 1)
    def _():
      mu = acc_r[0:1, :] / eg
      var = acc_r[1:2, :] / eg - mu * mu
      sc = g_r[...] * lax.rsqrt(var + 1e-5)
      sh = b_r[...] - mu * sc
      lo = acc_r[2:3, :] * sc + sh
      hi = acc_r[3:4, :] * sc + sh
      lo2 = jnp.minimum(lo, hi)
      hi2 = jnp.maximum(lo, hi)
      gm = jnp.maximum(lo2 * jax.nn.sigmoid(lo2), hi2 * jax.nn.sigmoid(hi2))
      stats_r[0:1, :] = sc
      stats_r[1:2, :] = sh
      stats_r[2:3, :] = gm
      stats_r[3:4, :] = jnp.zeros_like(gm)

  out = pl.pallas_call(
      body,
      grid=(nb,),
      in_specs=[
          pl.BlockSpec((be, DIM), lambda i: (i, 0)),
          pl.BlockSpec((be, DIM), lambda i: (i, 0)),
          pl.BlockSpec((be, DIM), lambda i: (i, 0)),
          pl.BlockSpec((DIM, HEADS * DIM), lambda i: (0, 0)),
          pl.BlockSpec((DIM, HEADS * DIM), lambda i: (0, 0)),
          pl.BlockSpec((HEADS * DIM, HP), lambda i: (0, 0)),
          pl.BlockSpec((HEADS * DIM, HP), lambda i: (0, 0)),
          pl.BlockSpec((1, HP), lambda i: (0, 0)),
          pl.BlockSpec((1, HP), lambda i: (0, 0)),
      ],
      out_specs=[
          pl.BlockSpec((be, HEADS * DIM), lambda i: (i, 0)),
          pl.BlockSpec((HP, be), lambda i: (0, i)),
          pl.BlockSpec((4, HP), lambda i: (0, 0)),
      ],
      out_shape=[
          jax.ShapeDtypeStruct((eg, HEADS * DIM), jnp.bfloat16),
          jax.ShapeDtypeStruct((HP, eg), _F32),
          jax.ShapeDtypeStruct((4, HP), _F32),
      ],
      scratch_shapes=[pltpu.VMEM((4, HP), _F32)],
  )(xi, xj, ea, w1, w2, att1_8, att2_8, g8, b8)
  return out


def _tc_msum(njq, wt4):
  eg = njq.shape[0]
  be = 1280
  nb = eg // be

  def body(nj_r, wt_r, out_r):
    nj = nj_r[...].astype(_F32)
    w4 = wt_r[...].T
    acc = nj[:, 0:DIM] * w4[:, 0:1]
    for h in range(1, HEADS):
      acc = acc + nj[:, h * DIM:(h + 1) * DIM] * w4[:, h:h + 1]
    out_r[...] = acc

  return pl.pallas_call(
      body,
      grid=(nb,),
      in_specs=[
          pl.BlockSpec((be, HEADS * DIM), lambda i: (i, 0)),
          pl.BlockSpec((HEADS, be), lambda i: (0, i)),
      ],
      out_specs=pl.BlockSpec((be, DIM), lambda i: (i, 0)),
      out_shape=jax.ShapeDtypeStruct((eg, DIM), _F32),
  )(njq, wt4)


def _tc_edge_update(nfi, nfj, ea, lf1, lf2, lf3, lfb, ls1, ls2, ls3, lsb,
                    g_row, b_row):
  eg = nfi.shape[0]
  be = 2000
  nb = eg // be

  def body(nfi_r, nfj_r, ea_r, lf1_r, lf2_r, lf3_r, lfb_r,
           ls1_r, ls2_r, ls3_r, lsb_r, g_r, b_r, m_r, stats_r, acc_r):
    step = pl.program_id(0)
    zi = (jnp.dot(nfi_r[...], lf1_r[...], preferred_element_type=_F32)
          + jnp.dot(nfj_r[...], lf2_r[...], preferred_element_type=_F32)
          + jnp.dot(ea_r[...], lf3_r[...], preferred_element_type=_F32)
          + lfb_r[...])
    zs = (jnp.dot(nfi_r[...], ls1_r[...], preferred_element_type=_F32)
          + jnp.dot(nfj_r[...], ls2_r[...], preferred_element_type=_F32)
          + jnp.dot(ea_r[...], ls3_r[...], preferred_element_type=_F32)
          + lsb_r[...])
    m = jax.nn.sigmoid(zi) * jax.nn.softplus(zs)
    m_r[...] = m
    s = jnp.sum(m, axis=0, keepdims=True)
    sq = jnp.sum(m * m, axis=0, keepdims=True)

    @pl.when(step == 0)
    def _():
      acc_r[0:1, :] = s
      acc_r[1:2, :] = sq

    @pl.when(step > 0)
    def _():
      acc_r[0:1, :] = acc_r[0:1, :] + s
      acc_r[1:2, :] = acc_r[1:2, :] + sq

    @pl.when(step == nb - 1)
    def _():
      mu = acc_r[0:1, :] / eg
      var = acc_r[1:2, :] / eg - mu * mu
      sc = g_r[...] * lax.rsqrt(var + 1e-5)
      stats_r[0:1, :] = sc
      stats_r[1:2, :] = b_r[...] - mu * sc

  return pl.pallas_call(
      body,
      grid=(nb,),
      in_specs=[pl.BlockSpec((be, DIM), lambda i: (i, 0))] * 3
      + [pl.BlockSpec((DIM, DIM), lambda i: (0, 0)),
         pl.BlockSpec((DIM, DIM), lambda i: (0, 0)),
         pl.BlockSpec((DIM, DIM), lambda i: (0, 0)),
         pl.BlockSpec((1, DIM), lambda i: (0, 0))] * 2
      + [pl.BlockSpec((1, DIM), lambda i: (0, 0)),
         pl.BlockSpec((1, DIM), lambda i: (0, 0))],
      out_specs=[
          pl.BlockSpec((be, DIM), lambda i: (i, 0)),
          pl.BlockSpec((2, DIM), lambda i: (0, 0)),
      ],
      out_shape=[
          jax.ShapeDtypeStruct((eg, DIM), _F32),
          jax.ShapeDtypeStruct((2, DIM), _F32),
      ],
      scratch_shapes=[pltpu.VMEM((2, DIM), _F32)],
  )(nfi, nfj, ea, lf1, lf2, lf3, lfb, ls1, ls2, ls3, lsb, g_row, b_row)


def _tc_axpb(base, m, scale_row, shift_row):
  n = base.shape[0]
  be = 2000
  nb = n // be

  def body(b_r, m_r, sc_r, sh_r, o_r):
    o_r[...] = b_r[...] + m_r[...] * sc_r[...] + sh_r[...]

  return pl.pallas_call(
      body,
      grid=(nb,),
      in_specs=[
          pl.BlockSpec((be, DIM), lambda i: (i, 0)),
          pl.BlockSpec((be, DIM), lambda i: (i, 0)),
          pl.BlockSpec((1, DIM), lambda i: (0, 0)),
          pl.BlockSpec((1, DIM), lambda i: (0, 0)),
      ],
      out_specs=pl.BlockSpec((be, DIM), lambda i: (i, 0)),
      out_shape=jax.ShapeDtypeStruct((n, DIM), _F32),
  )(base, m, scale_row, shift_row)


def _gcao_layer(x, edge_index, ea, w, att, bias, bnn_g, bnn_b,
                lf_w, lf_b, ls_w, ls_b, bne_g, bne_b):
  n = x.shape[0]
  idx_i = edge_index[0]
  idx_j = edge_index[1]

  w1 = w[:DIM]
  w2 = w[DIM:]
  a1 = att[0, :, :DIM]
  a2 = att[0, :, DIM:]
  eye = jnp.eye(HEADS, HP, dtype=_F32)
  att1_8 = jnp.einsum("hd,hp->hdp", a1, eye).reshape(HEADS * DIM, HP)
  att2_8 = jnp.einsum("hd,hp->hdp", a2, eye).reshape(HEADS * DIM, HP)
  g8 = jnp.concatenate([bnn_g, jnp.ones((HP - HEADS,), _F32)])[None, :]
  b8 = jnp.concatenate([bnn_b, jnp.zeros((HP - HEADS,), _F32)])[None, :]

  xi, xj = _sc_gather2(x, idx_i, idx_j)
  njq, alpha8, stats = _tc_edge_attn(xi, xj, ea, w1, w2, att1_8, att2_8,
                                     g8, b8)
  ctab = jnp.zeros((16, 16), _F32).at[0:12, :].set(
      jnp.repeat(stats[0:3, 0:HEADS].reshape(12, 1), 16, axis=1)).reshape(256)
  eg = njq.shape[0]
  w_f = _sc_seg_softmax(alpha8.reshape(-1), idx_i, ctab, n)
  msum = _tc_msum(njq, w_f.reshape(HEADS, eg))
  aggr = _sc_scatter_aggr(msum, idx_i, n)
  ones_row = jnp.ones((1, DIM), _F32)
  node_feat = _tc_axpb(x, aggr, ones_row, bias[None, :])

  nfi, nfj = _sc_gather2(node_feat, idx_i, idx_j)
  lf1 = lf_w[:, 0:DIM].T
  lf2 = lf_w[:, DIM:2 * DIM].T
  lf3 = lf_w[:, 2 * DIM:].T
  ls1 = ls_w[:, 0:DIM].T
  ls2 = ls_w[:, DIM:2 * DIM].T
  ls3 = ls_w[:, 2 * DIM:].T
  m_raw, mstats = _tc_edge_update(nfi, nfj, ea, lf1, lf2, lf3, lf_b[None, :],
                                  ls1, ls2, ls3, ls_b[None, :],
                                  bne_g[None, :], bne_b[None, :])
  edge_feat = _tc_axpb(ea, m_raw, mstats[0:1, :], mstats[1:2, :])
  return node_feat, edge_feat


def kernel(atom_feats, bond_attr, triplet_feats, edge_index, angle_index,
           ba_W, ba_att, ba_bias, ba_bnn_g, ba_bnn_b, ba_lf_w, ba_lf_b,
           ba_ls_w, ba_ls_b, ba_bne_g, ba_bne_b,
           bb_W, bb_att, bb_bias, bb_bnn_g, bb_bnn_b, bb_lf_w, bb_lf_b,
           bb_ls_w, bb_ls_b, bb_bne_g, bb_bne_b):
  bond, triplet = _gcao_layer(bond_attr, angle_index, triplet_feats,
                              ba_W, ba_att, ba_bias, ba_bnn_g, ba_bnn_b,
                              ba_lf_w, ba_lf_b, ba_ls_w, ba_ls_b,
                              ba_bne_g, ba_bne_b)
  atom, bond2 = _gcao_layer(atom_feats, edge_index, bond,
                            bb_W, bb_att, bb_bias, bb_bnn_g, bb_bnn_b,
                            bb_lf_w, bb_lf_b, bb_ls_w, bb_ls_b,
                            bb_bne_g, bb_bne_b)
  return (atom, bond2, triplet)
